# Initial kernel scaffold; baseline (speedup 1.0000x reference)
import jax
import jax.numpy as jnp
from jax.experimental import pallas as pl


def kernel(input_ids, amp, pitch, boundary):
    raise NotImplementedError("write your pallas kernel here")


# speedup vs baseline: 1.0474x; 1.0474x over previous
import functools

import jax
import jax.numpy as jnp
from jax import lax
from jax.experimental import pallas as pl
from jax.experimental.pallas import tpu as pltpu
from jax.experimental.pallas import tpu_sc as plsc

B, S = 64, 4096
L = 16
NCHUNK = S // L
NW = 32
ROWS_PER_W = B // NW
PAD = L
K = 5
W_AMP, W_PITCH = 0.8, 1.2
GAIN_UP, GAIN_DOWN = 1.5, 0.7


def _body(amp_hbm, pitch_hbm, bnd_hbm, out_hbm, a_v, p_v, b_v, sal_v, sm_v, o_v):
    cid = lax.axis_index("c")
    sid = lax.axis_index("s")
    wid = sid * 2 + cid

    iota = lax.broadcasted_iota(jnp.int32, (L,), 0)
    neg = jnp.full((L,), -1.0, jnp.float32)
    big = jnp.full((L,), 1e30, jnp.float32)
    zero = jnp.zeros((L,), jnp.float32)

    for rr in range(ROWS_PER_W):
        row = wid * ROWS_PER_W + rr
        base = row * S
        pltpu.sync_copy(amp_hbm.at[pl.ds(base, S)], a_v)
        pltpu.sync_copy(pitch_hbm.at[pl.ds(base, S)], p_v)
        pltpu.sync_copy(bnd_hbm.at[pl.ds(base, S)], b_v)

        def p1(j, carry):
            off = j * L
            sal = (W_AMP * a_v[pl.ds(off, L)] + W_PITCH * p_v[pl.ds(off, L)]) \
                + b_v[pl.ds(off, L)]
            sal_v[pl.ds(PAD + off, L)] = sal
            return carry
        lax.fori_loop(0, NCHUNK, p1, 0)

        sal_v[pl.ds(0, L)] = jnp.broadcast_to(sal_v[pl.ds(PAD, L)][0], (L,))
        sal_v[pl.ds(PAD + S, L)] = jnp.broadcast_to(
            sal_v[pl.ds(PAD + S - L, L)][L - 1], (L,))

        def p2(j, carry):
            m0, m1, m2, m3, m4, vmn, vsum = carry
            b2 = PAD + j * L - 2
            acc = sal_v[pl.ds(b2, L)]
            acc = acc + sal_v[pl.ds(b2 + 1, L)]
            acc = acc + sal_v[pl.ds(b2 + 2, L)]
            acc = acc + sal_v[pl.ds(b2 + 3, L)]
            acc = acc + sal_v[pl.ds(b2 + 4, L)]
            sm = acc / 5.0
            sm_v[pl.ds(j * L, L)] = sm
            vmn = jnp.minimum(vmn, sm)
            vsum = vsum + sm
            t = sm
            hi = jnp.maximum(m0, t); t = jnp.minimum(m0, t); m0 = hi
            hi = jnp.maximum(m1, t); t = jnp.minimum(m1, t); m1 = hi
            hi = jnp.maximum(m2, t); t = jnp.minimum(m2, t); m2 = hi
            hi = jnp.maximum(m3, t); t = jnp.minimum(m3, t); m3 = hi
            m4 = jnp.maximum(m4, t)
            return (m0, m1, m2, m3, m4, vmn, vsum)

        m0, m1, m2, m3, m4, vmn, vsum = lax.fori_loop(
            0, NCHUNK, p2, (neg, neg, neg, neg, neg, big, zero))

        mn = jnp.broadcast_to(jnp.min(vmn), (L,))
        s_all = jnp.broadcast_to(jnp.sum(vsum), (L,))

        w0, w1, w2, w3, w4 = m0, m1, m2, m3, m4
        s_top = zero
        g = zero
        for _ in range(K):
            g = jnp.broadcast_to(jnp.max(w0), (L,))
            s_top = s_top + g
            sel = iota == plsc.all_reduce_ffs(w0 == g)
            w0 = jnp.where(sel, w1, w0)
            w1 = jnp.where(sel, w2, w1)
            w2 = jnp.where(sel, w3, w2)
            w3 = jnp.where(sel, w4, w3)
            w4 = jnp.where(sel, neg, w4)
        mx = jnp.broadcast_to(jnp.max(m0), (L,))
        thr = g

        scale = 1.0 / (mx - mn + 1e-8)
        mu = (GAIN_DOWN * scale * (s_all - float(S) * mn)
              + (GAIN_UP - GAIN_DOWN) * scale * (s_top - float(K) * mn)) \
            / float(S)
        c_up = GAIN_UP * scale * mu
        c_dn = GAIN_DOWN * scale * mu

        def p3(j, carry):
            sm = sm_v[pl.ds(j * L, L)]
            o_v[pl.ds(j * L, L)] = (sm - mn) * jnp.where(sm >= thr, c_up, c_dn)
            return carry
        lax.fori_loop(0, NCHUNK, p3, 0)

        pltpu.sync_copy(o_v, out_hbm.at[pl.ds(base, S)])


@jax.jit
def _bridge(amp, pitch, boundary):
    mesh = plsc.VectorSubcoreMesh(core_axis_name="c", subcore_axis_name="s")
    kfn = pl.kernel(
        _body,
        out_type=jax.ShapeDtypeStruct((B * S,), jnp.float32),
        mesh=mesh,
        scratch_types=[
            pltpu.VMEM((S,), jnp.float32),
            pltpu.VMEM((S,), jnp.float32),
            pltpu.VMEM((S,), jnp.float32),
            pltpu.VMEM((S + 2 * PAD,), jnp.float32),
            pltpu.VMEM((S,), jnp.float32),
            pltpu.VMEM((S,), jnp.float32),
        ],
        compiler_params=pltpu.CompilerParams(needs_layout_passes=False),
    )
    out = kfn(amp.reshape(B * S), pitch.reshape(B * S), boundary.reshape(B * S))
    return out.reshape(B, S)


def kernel(input_ids, amp, pitch, boundary):
    del input_ids
    return _bridge(amp, pitch, boundary)
